# baseline (speedup 1.0000x reference)
import jax
import jax.numpy as jnp
from jax.experimental import pallas as pl
from jax.experimental.pallas import tpu as pltpu

N = 4096
NUM_BINS = 3
BIN = N // NUM_BINS
K1 = BIN
K2 = 2 * BIN
B = 512
NBLK = N // B

_MSB = 0x80000000


def _select_two(ukey, ka0, kb0):
    ones = jnp.ones(ukey.shape, jnp.int32)

    def pass_body(i, carry):
        ka, va, ma, kb, vb, mb = carry
        shift = jnp.uint32(28) - jnp.uint32(4) * i.astype(jnp.uint32)
        d = (jax.lax.shift_right_logical(ukey, shift) &
             jnp.uint32(15)).astype(jnp.int32)
        e = [jnp.where(d == j, 1, 0) for j in range(15)]

        def advance(k, val, m):
            c = [jnp.sum(m * e[j]) for j in range(15)]
            prefix = [c[0]]
            for j in range(1, 15):
                prefix.append(prefix[-1] + c[j])
            sel = jnp.int32(0)
            for j in range(15):
                sel = sel + jnp.where(k >= prefix[j], 1, 0)
            dec = jnp.int32(0)
            for j in range(15):
                dec = dec + jnp.where(sel == j + 1, prefix[j], 0)
            k = k - dec
            m = m * jnp.where(d == sel, 1, 0)
            val = val | jax.lax.shift_left(sel.astype(jnp.uint32), shift)
            return k, val, m

        ka, va, ma = advance(ka, va, ma)
        kb, vb, mb = advance(kb, vb, mb)
        return ka, va, ma, kb, vb, mb

    init = (jnp.int32(ka0), jnp.uint32(0), ones,
            jnp.int32(kb0), jnp.uint32(0), ones)
    _, va, _, _, vb, _ = jax.lax.fori_loop(0, 8, pass_body, init)
    return va, vb


def _col_params(w, s, t1, t2):
    q_min = jnp.where(s <= t1, -2.0,
                      jnp.where(s <= t2, -8.0, -32.0)).astype(jnp.float32)
    q_max = -q_min - 1.0
    min_vals = jnp.min(w, axis=0, keepdims=True)
    max_vals = jnp.max(w, axis=0, keepdims=True)
    scale = (max_vals - min_vals) / (q_max - q_min)
    scale = jnp.where(jnp.abs(scale) < 1e-6, jnp.float32(1e-6), scale)
    inv = 1.0 / scale
    zp = jnp.clip(jnp.round(q_min - min_vals / scale), q_min, q_max)
    return scale, inv, zp


def _quant_rows(w, scale, inv, zp):
    q = jnp.clip(jnp.round(w * inv) + zp, -128.0, 127.0)
    return (q - zp) * scale


def _body(s2d_ref, s_blk_ref, w_hbm, out_hbm,
          wbuf0, wbuf1, wbuf2, obuf0, obuf1, si0, si1, si2, so0, so1):
    wbufs = (wbuf0, wbuf1, wbuf2)
    sis = (si0, si1, si2)
    obufs = (obuf0, obuf1)
    sos = (so0, so1)

    def in_copy(b, buf, sem):
        return pltpu.make_async_copy(
            w_hbm.at[:, pl.ds(b * B, B)], buf, sem)

    def out_copy(b, buf, sem, h):
        return pltpu.make_async_copy(
            buf.at[pl.ds(h * (N // 2), N // 2), :],
            out_hbm.at[pl.ds(h * (N // 2), N // 2), pl.ds(b * B, B)], sem)

    in_copy(0, wbuf0, si0).start()
    in_copy(1, wbuf1, si1).start()
    in_copy(2, wbuf2, si2).start()

    s = s2d_ref[...]
    u = jax.lax.bitcast_convert_type(s, jnp.uint32)
    msb = jnp.uint32(_MSB)
    ukey = jnp.where(u < msb, u | msb, ~u)

    def unmap(v):
        bits = jnp.where(v >= msb, v ^ msb, ~v)
        return jax.lax.bitcast_convert_type(bits, jnp.float32)

    va, vb = _select_two(ukey, K1, K2)
    t1 = unmap(va)
    t2 = unmap(vb)

    H = N // 2
    for b in range(NBLK):
        ws, si = wbufs[b % 3], sis[b % 3]
        ob, so = obufs[b % 2], sos[b % 2]
        in_copy(b, ws, si).wait()
        scale, inv, zp = _col_params(ws[...], s_blk_ref[b], t1, t2)
        if b >= 2:
            out_copy(b - 2, ob, so, 0).wait()
            out_copy(b - 2, ob, so, 1).wait()
        ob[pl.ds(0, H), :] = _quant_rows(ws[pl.ds(0, H), :], scale, inv, zp)
        out_copy(b, ob, so, 0).start()
        ob[pl.ds(H, H), :] = _quant_rows(ws[pl.ds(H, H), :], scale, inv, zp)
        out_copy(b, ob, so, 1).start()
        if b + 3 < NBLK:
            in_copy(b + 3, ws, si).start()

    for b in (NBLK - 2, NBLK - 1):
        out_copy(b, obufs[b % 2], sos[b % 2], 0).wait()
        out_copy(b, obufs[b % 2], sos[b % 2], 1).wait()


def kernel(weight, scores):
    s2d = scores.reshape(32, 128)
    s_blk = scores.reshape(NBLK, 1, B)
    out = pl.pallas_call(
        _body,
        in_specs=[
            pl.BlockSpec((32, 128), lambda: (0, 0)),
            pl.BlockSpec((NBLK, 1, B), lambda: (0, 0, 0)),
            pl.BlockSpec(memory_space=pl.ANY),
        ],
        out_specs=pl.BlockSpec(memory_space=pl.ANY),
        out_shape=jax.ShapeDtypeStruct((N, N), jnp.float32),
        scratch_shapes=[
            pltpu.VMEM((N, B), jnp.float32),
            pltpu.VMEM((N, B), jnp.float32),
            pltpu.VMEM((N, B), jnp.float32),
            pltpu.VMEM((N, B), jnp.float32),
            pltpu.VMEM((N, B), jnp.float32),
            pltpu.SemaphoreType.DMA,
            pltpu.SemaphoreType.DMA,
            pltpu.SemaphoreType.DMA,
            pltpu.SemaphoreType.DMA,
            pltpu.SemaphoreType.DMA,
        ],
    )(s2d, s_blk, weight)
    return out


# speedup vs baseline: 1.0013x; 1.0013x over previous
import jax
import jax.numpy as jnp
from jax.experimental import pallas as pl
from jax.experimental.pallas import tpu as pltpu

N = 4096
NUM_BINS = 3
BIN = N // NUM_BINS
K1 = BIN
K2 = 2 * BIN
B = 512
NBLK = N // B

_MSB = 0x80000000


def _select_two(ukey, ka0, kb0):
    ones = jnp.ones(ukey.shape, jnp.int32)

    def pass_body(i, carry):
        ka, va, ma, kb, vb, mb = carry
        shift = jnp.uint32(28) - jnp.uint32(4) * i.astype(jnp.uint32)
        d = (jax.lax.shift_right_logical(ukey, shift) &
             jnp.uint32(15)).astype(jnp.int32)
        e = [jnp.where(d == j, 1, 0) for j in range(15)]

        def advance(k, val, m):
            c = [jnp.sum(m * e[j]) for j in range(15)]
            prefix = [c[0]]
            for j in range(1, 15):
                prefix.append(prefix[-1] + c[j])
            sel = jnp.int32(0)
            for j in range(15):
                sel = sel + jnp.where(k >= prefix[j], 1, 0)
            dec = jnp.int32(0)
            for j in range(15):
                dec = dec + jnp.where(sel == j + 1, prefix[j], 0)
            k = k - dec
            m = m * jnp.where(d == sel, 1, 0)
            val = val | jax.lax.shift_left(sel.astype(jnp.uint32), shift)
            return k, val, m

        ka, va, ma = advance(ka, va, ma)
        kb, vb, mb = advance(kb, vb, mb)
        return ka, va, ma, kb, vb, mb

    init = (jnp.int32(ka0), jnp.uint32(0), ones,
            jnp.int32(kb0), jnp.uint32(0), ones)
    _, va, _, _, vb, _ = jax.lax.fori_loop(0, 8, pass_body, init)
    return va, vb


def _quant_block(w, s, t1, t2):
    q_min = jnp.where(s <= t1, -2.0,
                      jnp.where(s <= t2, -8.0, -32.0)).astype(jnp.float32)
    q_max = -q_min - 1.0
    min_vals = jnp.min(w, axis=0, keepdims=True)
    max_vals = jnp.max(w, axis=0, keepdims=True)
    scale = (max_vals - min_vals) / (q_max - q_min)
    scale = jnp.where(jnp.abs(scale) < 1e-6, jnp.float32(1e-6), scale)
    inv = 1.0 / scale
    zp = jnp.clip(jnp.round(q_min - min_vals / scale), q_min, q_max)
    q = jnp.clip(jnp.round(w * inv) + zp, -128.0, 127.0)
    return (q - zp) * scale


def _body(s2d_ref, s_blk_ref, w_hbm, out_hbm,
          wbuf0, wbuf1, wbuf2, obuf0, obuf1, si0, si1, si2, so0, so1):
    wbufs = (wbuf0, wbuf1, wbuf2)
    sis = (si0, si1, si2)
    obufs = (obuf0, obuf1)
    sos = (so0, so1)

    def in_copy(b, buf, sem):
        return pltpu.make_async_copy(
            w_hbm.at[:, pl.ds(b * B, B)], buf, sem)

    def out_copy(b, buf, sem):
        return pltpu.make_async_copy(
            buf, out_hbm.at[:, pl.ds(b * B, B)], sem)

    in_copy(0, wbuf0, si0).start()
    in_copy(1, wbuf1, si1).start()
    in_copy(2, wbuf2, si2).start()

    s = s2d_ref[...]
    u = jax.lax.bitcast_convert_type(s, jnp.uint32)
    msb = jnp.uint32(_MSB)
    ukey = jnp.where(u < msb, u | msb, ~u)

    def unmap(v):
        bits = jnp.where(v >= msb, v ^ msb, ~v)
        return jax.lax.bitcast_convert_type(bits, jnp.float32)

    va, vb = _select_two(ukey, K1, K2)
    t1 = unmap(va)
    t2 = unmap(vb)

    for b in range(NBLK):
        ws, si = wbufs[b % 3], sis[b % 3]
        ob, so = obufs[b % 2], sos[b % 2]
        in_copy(b, ws, si).wait()
        if b >= 2:
            out_copy(b - 2, ob, so).wait()
        ob[...] = _quant_block(ws[...], s_blk_ref[b], t1, t2)
        out_copy(b, ob, so).start()
        if b + 3 < NBLK:
            in_copy(b + 3, ws, si).start()

    out_copy(NBLK - 2, obufs[0], sos[0]).wait()
    out_copy(NBLK - 1, obufs[1], sos[1]).wait()


def kernel(weight, scores):
    s2d = scores.reshape(32, 128)
    s_blk = scores.reshape(NBLK, 1, B)
    out = pl.pallas_call(
        _body,
        in_specs=[
            pl.BlockSpec((32, 128), lambda: (0, 0)),
            pl.BlockSpec((NBLK, 1, B), lambda: (0, 0, 0)),
            pl.BlockSpec(memory_space=pl.ANY),
        ],
        out_specs=pl.BlockSpec(memory_space=pl.ANY),
        out_shape=jax.ShapeDtypeStruct((N, N), jnp.float32),
        scratch_shapes=[
            pltpu.VMEM((N, B), jnp.float32),
            pltpu.VMEM((N, B), jnp.float32),
            pltpu.VMEM((N, B), jnp.float32),
            pltpu.VMEM((N, B), jnp.float32),
            pltpu.VMEM((N, B), jnp.float32),
            pltpu.SemaphoreType.DMA,
            pltpu.SemaphoreType.DMA,
            pltpu.SemaphoreType.DMA,
            pltpu.SemaphoreType.DMA,
            pltpu.SemaphoreType.DMA,
        ],
    )(s2d, s_blk, weight)
    return out
